# baseline (speedup 1.0000x reference)
import functools

import jax
import jax.numpy as jnp
from jax import lax
from jax.experimental import pallas as pl
from jax.experimental.pallas import tpu as pltpu
from jax.experimental.pallas import tpu_sc as plsc

_NC = 2
_NS = 16
_NW = _NC * _NS


def _gather_fn(B, D, b_per_w, chunk, n_chunks):
    mesh = plsc.VectorSubcoreMesh(core_axis_name="c", subcore_axis_name="s")

    @functools.partial(
        pl.kernel,
        mesh=mesh,
        out_type=jax.ShapeDtypeStruct((B, D), jnp.float32),
        compiler_params=pltpu.CompilerParams(use_tc_tiling_on_sc=False),
        scratch_types=[
            pltpu.VMEM((chunk,), jnp.int32),
            pltpu.VMEM((chunk, D), jnp.float32),
            pltpu.SemaphoreType.DMA,
        ],
    )
    def k(table_hbm, idx_hbm, out_hbm, idx_v, rows_v, sem):
        wid = lax.axis_index("s") * _NC + lax.axis_index("c")
        base = wid * b_per_w

        def body(i, carry):
            off = base + i * chunk
            pltpu.sync_copy(idx_hbm.at[pl.ds(off, chunk)], idx_v)
            pltpu.async_copy(table_hbm.at[idx_v], rows_v, sem).wait()
            pltpu.sync_copy(rows_v, out_hbm.at[pl.ds(off, chunk)])
            return carry

        lax.fori_loop(0, n_chunks, body, 0)

    return k


def kernel(input_feat, embed_weight):
    B = input_feat.shape[0] * input_feat.shape[1]
    D = embed_weight.shape[1]
    idx = input_feat.reshape(-1).astype(jnp.int32)
    b_per_w = B // _NW
    chunk = 1024
    n_chunks = b_per_w // chunk
    assert b_per_w % chunk == 0 and B % _NW == 0
    fn = _gather_fn(B, D, b_per_w, chunk, n_chunks)
    return fn(embed_weight, idx)


# speedup vs baseline: 1.0162x; 1.0162x over previous
import functools

import jax
import jax.numpy as jnp
from jax import lax
from jax.experimental import pallas as pl
from jax.experimental.pallas import tpu as pltpu
from jax.experimental.pallas import tpu_sc as plsc

_NC = 2
_NS = 16
_NW = _NC * _NS
_NBUF = 3
_LEAD = 2


def _gather_fn(B, D, b_per_w, chunk, n_chunks):
    mesh = plsc.VectorSubcoreMesh(core_axis_name="c", subcore_axis_name="s")

    @functools.partial(
        pl.kernel,
        mesh=mesh,
        out_type=jax.ShapeDtypeStruct((B, D), jnp.float32),
        compiler_params=pltpu.CompilerParams(use_tc_tiling_on_sc=False),
        scratch_types=[
            pltpu.VMEM((b_per_w,), jnp.int32),
            pltpu.VMEM((_NBUF, chunk, D), jnp.float32),
            [pltpu.SemaphoreType.DMA] * _NBUF,
            [pltpu.SemaphoreType.DMA] * _NBUF,
        ],
    )
    def k(table_hbm, idx_hbm, out_hbm, idx_v, rows_v, gsems, osems):
        wid = lax.axis_index("s") * _NC + lax.axis_index("c")
        base = wid * b_per_w

        pltpu.sync_copy(idx_hbm.at[pl.ds(base, b_per_w)], idx_v)

        def start_gather(j):
            b = j % _NBUF
            return pltpu.async_copy(
                table_hbm.at[idx_v.at[pl.ds(j * chunk, chunk)]],
                rows_v.at[b],
                gsems[b],
            )

        def start_out(j):
            b = j % _NBUF
            return pltpu.async_copy(
                rows_v.at[b],
                out_hbm.at[pl.ds(base + j * chunk, chunk)],
                osems[b],
            )

        g_handles = [None] * n_chunks
        o_handles = [None] * n_chunks
        for j in range(_LEAD):
            g_handles[j] = start_gather(j)
        for i in range(n_chunks):
            g_handles[i].wait()
            o_handles[i] = start_out(i)
            j = i + _LEAD
            if j < n_chunks:
                if j - _NBUF >= 0:
                    o_handles[j - _NBUF].wait()
                g_handles[j] = start_gather(j)
        for i in range(max(0, n_chunks - _NBUF), n_chunks):
            o_handles[i].wait()

    return k


def kernel(input_feat, embed_weight):
    B = input_feat.shape[0] * input_feat.shape[1]
    D = embed_weight.shape[1]
    idx = input_feat.reshape(-1).astype(jnp.int32)
    b_per_w = B // _NW
    chunk = 832
    n_chunks = b_per_w // chunk
    assert b_per_w % chunk == 0 and B % _NW == 0
    fn = _gather_fn(B, D, b_per_w, chunk, n_chunks)
    return fn(embed_weight, idx)
